# Initial kernel scaffold; baseline (speedup 1.0000x reference)
import jax
import jax.numpy as jnp
from jax.experimental import pallas as pl


def kernel(inputs, table):
    raise NotImplementedError("write your pallas kernel here")


# speedup vs baseline: 1.7738x; 1.7738x over previous
import functools

import jax
import jax.numpy as jnp
from jax import lax
from jax.experimental import pallas as pl
from jax.experimental.pallas import tpu as pltpu
from jax.experimental.pallas import tpu_sc as plsc

NC, NS = 2, 16
NW = NC * NS
D = 512
L = 64
C = 64


def _pos_body(out_ref):
    pos = lax.broadcasted_iota(jnp.int32, (L, D), 0).astype(jnp.float32)
    d = lax.broadcasted_iota(jnp.int32, (L, D), 1)
    k2 = ((d // 2) * 2).astype(jnp.float32)
    freq = jnp.exp(k2 * (-jnp.log(10000.0) / D))
    angle = pos * freq
    out_ref[...] = jnp.where(d % 2 == 0, jnp.cos(angle), jnp.sin(angle))


@jax.jit
def _pos_table():
    return pl.pallas_call(
        _pos_body,
        out_shape=jax.ShapeDtypeStruct((L, D), jnp.float32),
    )()


@functools.partial(jax.jit, static_argnames=("n_rows",))
def _sc_gather(idx, table, pos, *, n_rows):
    b_per_w = n_rows // NW
    n_chunks = b_per_w // C

    def body(idx_hbm, table_hbm, pos_hbm, out_hbm, idx_v, rows_v, pos_v, sem):
        wid = lax.axis_index("s") * NC + lax.axis_index("c")
        base = wid * b_per_w
        pltpu.sync_copy(pos_hbm, pos_v)

        def chunk(g, carry):
            off = base + g * C
            pltpu.sync_copy(idx_hbm.at[pl.ds(off, C)], idx_v)
            pltpu.async_copy(table_hbm.at[idx_v], rows_v, sem).wait()

            def row(r, c2):
                for j in range(D // 16):
                    sl = pl.ds(j * 16, 16)
                    rows_v[r, sl] = rows_v[r, sl] + pos_v[r, sl]
                return c2

            lax.fori_loop(0, C, row, 0)
            pltpu.sync_copy(rows_v, out_hbm.at[pl.ds(off, C)])
            return carry

        lax.fori_loop(0, n_chunks, chunk, 0)

    return pl.kernel(
        body,
        out_type=jax.ShapeDtypeStruct((n_rows, D), jnp.float32),
        mesh=plsc.VectorSubcoreMesh(core_axis_name="c", subcore_axis_name="s"),
        scratch_types=[
            pltpu.VMEM((C,), jnp.int32),
            pltpu.VMEM((C, D), jnp.float32),
            pltpu.VMEM((C, D), jnp.float32),
            pltpu.SemaphoreType.DMA,
        ],
    )(idx, table, pos)


def kernel(inputs, table):
    batch, seq = inputs.shape
    idx = inputs.reshape(-1).astype(jnp.int32)
    pos = _pos_table()
    out = _sc_gather(idx, table, pos, n_rows=batch * seq)
    return out.reshape(batch, seq, D)


# speedup vs baseline: 2.8994x; 1.6346x over previous
import functools

import jax
import jax.numpy as jnp
from jax import lax
from jax.experimental import pallas as pl
from jax.experimental.pallas import tpu as pltpu
from jax.experimental.pallas import tpu_sc as plsc

NC, NS = 2, 16
NW = NC * NS
D = 512
L = 64
C = 64


def _pos_body(out_ref):
    pos = lax.broadcasted_iota(jnp.int32, (L, D), 0).astype(jnp.float32)
    d = lax.broadcasted_iota(jnp.int32, (L, D), 1)
    k2 = ((d // 2) * 2).astype(jnp.float32)
    freq = jnp.exp(k2 * (-jnp.log(10000.0) / D))
    angle = pos * freq
    out_ref[...] = jnp.where(d % 2 == 0, jnp.cos(angle), jnp.sin(angle))


@jax.jit
def _pos_table():
    return pl.pallas_call(
        _pos_body,
        out_shape=jax.ShapeDtypeStruct((L, D), jnp.float32),
    )()


@functools.partial(jax.jit, static_argnames=("n_rows",))
def _sc_gather(idx, table, pos, *, n_rows):
    b_per_w = n_rows // NW
    n_chunks = b_per_w // C

    def body(idx_hbm, table_hbm, pos_hbm, out_hbm,
             idx0, idx1, rows0, rows1, pos_v,
             isem0, isem1, gsem0, gsem1, osem0, osem1):
        idxs, rows = [idx0, idx1], [rows0, rows1]
        isems, gsems, osems = [isem0, isem1], [gsem0, gsem1], [osem0, osem1]
        wid = lax.axis_index("s") * NC + lax.axis_index("c")
        base = wid * b_per_w
        pltpu.sync_copy(pos_hbm, pos_v)

        def idx_copy(g, b):
            return pltpu.make_async_copy(
                idx_hbm.at[pl.ds(base + g * C, C)], idxs[b], isems[b])

        def gather_copy(b):
            return pltpu.make_async_copy(table_hbm.at[idxs[b]], rows[b], gsems[b])

        def out_copy(g, b):
            return pltpu.make_async_copy(
                rows[b], out_hbm.at[pl.ds(base + g * C, C)], osems[b])

        def add_pos(b):
            def row(r, c2):
                for j in range(D // 16):
                    sl = pl.ds(j * 16, 16)
                    rows[b][r, sl] = rows[b][r, sl] + pos_v[r, sl]
                return c2

            lax.fori_loop(0, C, row, 0)

        for b in range(2):
            idx_copy(b, b).start()
            idx_copy(b, b).wait()
            gather_copy(b).start()

        def step(g2, carry):
            g = g2 * 2
            for b in range(2):
                gc = g + b
                gather_copy(b).wait()

                @pl.when(gc + 2 < n_chunks)
                def _():
                    idx_copy(gc + 2, b).start()

                add_pos(b)
                out_copy(gc, b).start()
                out_copy(gc, b).wait()

                @pl.when(gc + 2 < n_chunks)
                def _():
                    idx_copy(gc + 2, b).wait()
                    gather_copy(b).start()

            return carry

        lax.fori_loop(0, n_chunks // 2, step, 0)

    return pl.kernel(
        body,
        out_type=jax.ShapeDtypeStruct((n_rows, D), jnp.float32),
        mesh=plsc.VectorSubcoreMesh(core_axis_name="c", subcore_axis_name="s"),
        scratch_types=[
            pltpu.VMEM((C,), jnp.int32),
            pltpu.VMEM((C,), jnp.int32),
            pltpu.VMEM((C, D), jnp.float32),
            pltpu.VMEM((C, D), jnp.float32),
            pltpu.VMEM((C, D), jnp.float32),
            pltpu.SemaphoreType.DMA,
            pltpu.SemaphoreType.DMA,
            pltpu.SemaphoreType.DMA,
            pltpu.SemaphoreType.DMA,
            pltpu.SemaphoreType.DMA,
            pltpu.SemaphoreType.DMA,
        ],
    )(idx, table, pos)


def kernel(inputs, table):
    batch, seq = inputs.shape
    idx = inputs.reshape(-1).astype(jnp.int32)
    pos = _pos_table()
    out = _sc_gather(idx, table, pos, n_rows=batch * seq)
    return out.reshape(batch, seq, D)
